# Initial kernel scaffold; baseline (speedup 1.0000x reference)
import jax
import jax.numpy as jnp
from jax.experimental import pallas as pl


def kernel(p, features, labels):
    raise NotImplementedError("write your pallas kernel here")


# speedup vs baseline: 20.3817x; 20.3817x over previous
import jax
import jax.numpy as jnp
from jax.experimental import pallas as pl

N = 8192
D = 32
NSAMPLE = 16
TEMP = 0.1
EPS = 1e-7
BQ = 128

_INF = 3.0e38


def _body(pb_ref, paT_ref, fb_ref, faT_ref, labc_ref, labr_ref,
          ls_ref, ms_ref):
    i = pl.program_id(0)

    pb = pb_ref[...]
    paT = paT_ref[...]

    mm = jnp.dot(pb.astype(jnp.bfloat16), paT.astype(jnp.bfloat16),
                 preferred_element_type=jnp.float32)
    d2 = (jnp.sum(pb * pb, axis=1, keepdims=True)
          + jnp.sum(paT * paT, axis=0, keepdims=True)
          - 2.0 * mm)

    t0 = jnp.min(d2, axis=1, keepdims=True)
    t = t0
    for _ in range(NSAMPLE):
        t = jnp.min(jnp.where(d2 > t, d2, _INF), axis=1, keepdims=True)
    nmask = jnp.logical_and(d2 <= t, d2 > t0)

    fb = fb_ref[...]
    faT = faT_ref[...]
    fd2 = (jnp.sum(fb * fb, axis=1, keepdims=True)
           + jnp.sum(faT * faT, axis=0, keepdims=True)
           - 2.0 * jnp.dot(fb, faT, preferred_element_type=jnp.float32,
                           precision=jax.lax.Precision.HIGHEST))
    row = i * BQ + jax.lax.broadcasted_iota(jnp.int32, (BQ, N), 0)
    col = jax.lax.broadcasted_iota(jnp.int32, (BQ, N), 1)
    fd2 = jnp.where(row == col, 0.0, fd2)
    dist = jnp.sqrt(jnp.maximum(fd2, 0.0) + EPS)

    dmin = jnp.min(jnp.where(nmask, dist, _INF), axis=1, keepdims=True)
    e = jnp.where(nmask, jnp.exp((dmin - dist) * (1.0 / TEMP)), 0.0)
    eq = labc_ref[...] == labr_ref[...]

    pos = jnp.sum(jnp.where(eq, e, 0.0), axis=1, keepdims=True)
    neg = jnp.sum(e, axis=1, keepdims=True)
    pcnt = jnp.sum(jnp.where(jnp.logical_and(eq, nmask), 1.0, 0.0),
                   axis=1, keepdims=True)
    pm = jnp.logical_and(pcnt > 0.5, pcnt < NSAMPLE - 0.5).astype(jnp.float32)
    lpp = -jnp.log(pos / neg + EPS)

    pls = jnp.sum(lpp * pm, axis=0, keepdims=True)
    pms = jnp.sum(pm, axis=0, keepdims=True)

    @pl.when(i == 0)
    def _():
        ls_ref[...] = pls
        ms_ref[...] = pms

    @pl.when(i > 0)
    def _():
        ls_ref[...] += pls
        ms_ref[...] += pms


def kernel(p, features, labels):
    p = p.astype(jnp.float32)
    features = features.astype(jnp.float32)
    pp = jnp.pad(p, ((0, 0), (0, 5)))
    paT = pp.T
    faT = features.T
    lab = labels.astype(jnp.float32)
    labc = lab.reshape(N, 1)
    labr = lab.reshape(1, N)

    ls, ms = pl.pallas_call(
        _body,
        grid=(N // BQ,),
        in_specs=[
            pl.BlockSpec((BQ, 8), lambda i: (i, 0)),
            pl.BlockSpec((8, N), lambda i: (0, 0)),
            pl.BlockSpec((BQ, D), lambda i: (i, 0)),
            pl.BlockSpec((D, N), lambda i: (0, 0)),
            pl.BlockSpec((BQ, 1), lambda i: (i, 0)),
            pl.BlockSpec((1, N), lambda i: (0, 0)),
        ],
        out_specs=[
            pl.BlockSpec((1, 1), lambda i: (0, 0)),
            pl.BlockSpec((1, 1), lambda i: (0, 0)),
        ],
        out_shape=[
            jax.ShapeDtypeStruct((1, 1), jnp.float32),
            jax.ShapeDtypeStruct((1, 1), jnp.float32),
        ],
        interpret=_INTERPRET,
    )(pp, paT, features, faT, labc, labr)

    return (ls[0, 0] / jnp.maximum(ms[0, 0], 1.0)).astype(jnp.float32)


_INTERPRET = False


# speedup vs baseline: 35.2471x; 1.7293x over previous
import jax
import jax.numpy as jnp
from jax.experimental import pallas as pl

N = 8192
D = 32
NSAMPLE = 16
TEMP = 0.1
EPS = 1e-7
BQ = 128

_INF = 3.0e38


def _body(pb_ref, paT_ref, fb_ref, faT_ref, labc_ref, labr_ref,
          ls_ref, ms_ref):
    i = pl.program_id(0)

    pb = pb_ref[...]
    paT = paT_ref[...]

    mm = jnp.dot(pb.astype(jnp.bfloat16), paT.astype(jnp.bfloat16),
                 preferred_element_type=jnp.float32)
    d2 = (jnp.sum(pb * pb, axis=1, keepdims=True)
          + jnp.sum(paT * paT, axis=0, keepdims=True)
          - 2.0 * mm)

    nch = 16
    w = N // nch
    chunks = [d2[:, c * w:(c + 1) * w] for c in range(nch)]
    m1 = chunks[0]
    for c in chunks[1:]:
        m1 = jnp.minimum(m1, c)
    m2 = jnp.full((BQ, w), _INF, jnp.float32)
    for c in chunks:
        m2 = jnp.minimum(m2, jnp.where(c > m1, c, _INF))
    red = jnp.concatenate([m1, m2], axis=1)
    t0 = jnp.min(m1, axis=1, keepdims=True)
    t = t0
    for _ in range(NSAMPLE):
        t = jnp.min(jnp.where(red > t, red, _INF), axis=1, keepdims=True)
    nmask = jnp.logical_and(d2 <= t, d2 > t0)

    fb = fb_ref[...]
    faT = faT_ref[...]
    fd2 = (jnp.sum(fb * fb, axis=1, keepdims=True)
           + jnp.sum(faT * faT, axis=0, keepdims=True)
           - 2.0 * jnp.dot(fb, faT, preferred_element_type=jnp.float32,
                           precision=jax.lax.Precision.HIGHEST))
    row = i * BQ + jax.lax.broadcasted_iota(jnp.int32, (BQ, N), 0)
    col = jax.lax.broadcasted_iota(jnp.int32, (BQ, N), 1)
    fd2 = jnp.where(row == col, 0.0, fd2)
    dist = jnp.sqrt(jnp.maximum(fd2, 0.0) + EPS)

    dmin = jnp.min(jnp.where(nmask, dist, _INF), axis=1, keepdims=True)
    e = jnp.where(nmask, jnp.exp((dmin - dist) * (1.0 / TEMP)), 0.0)
    eq = labc_ref[...] == labr_ref[...]

    pos = jnp.sum(jnp.where(eq, e, 0.0), axis=1, keepdims=True)
    neg = jnp.sum(e, axis=1, keepdims=True)
    pcnt = jnp.sum(jnp.where(jnp.logical_and(eq, nmask), 1.0, 0.0),
                   axis=1, keepdims=True)
    pm = jnp.logical_and(pcnt > 0.5, pcnt < NSAMPLE - 0.5).astype(jnp.float32)
    lpp = -jnp.log(pos / neg + EPS)

    pls = jnp.sum(lpp * pm, axis=0, keepdims=True)
    pms = jnp.sum(pm, axis=0, keepdims=True)

    @pl.when(i == 0)
    def _():
        ls_ref[...] = pls
        ms_ref[...] = pms

    @pl.when(i > 0)
    def _():
        ls_ref[...] += pls
        ms_ref[...] += pms


def kernel(p, features, labels):
    p = p.astype(jnp.float32)
    features = features.astype(jnp.float32)
    pp = jnp.pad(p, ((0, 0), (0, 5)))
    paT = pp.T
    faT = features.T
    lab = labels.astype(jnp.float32)
    labc = lab.reshape(N, 1)
    labr = lab.reshape(1, N)

    ls, ms = pl.pallas_call(
        _body,
        grid=(N // BQ,),
        in_specs=[
            pl.BlockSpec((BQ, 8), lambda i: (i, 0)),
            pl.BlockSpec((8, N), lambda i: (0, 0)),
            pl.BlockSpec((BQ, D), lambda i: (i, 0)),
            pl.BlockSpec((D, N), lambda i: (0, 0)),
            pl.BlockSpec((BQ, 1), lambda i: (i, 0)),
            pl.BlockSpec((1, N), lambda i: (0, 0)),
        ],
        out_specs=[
            pl.BlockSpec((1, 1), lambda i: (0, 0)),
            pl.BlockSpec((1, 1), lambda i: (0, 0)),
        ],
        out_shape=[
            jax.ShapeDtypeStruct((1, 1), jnp.float32),
            jax.ShapeDtypeStruct((1, 1), jnp.float32),
        ],
        interpret=_INTERPRET,
    )(pp, paT, features, faT, labc, labr)

    return (ls[0, 0] / jnp.maximum(ms[0, 0], 1.0)).astype(jnp.float32)


_INTERPRET = False


# speedup vs baseline: 37.5640x; 1.0657x over previous
import jax
import jax.numpy as jnp
from jax.experimental import pallas as pl

N = 8192
D = 32
NSAMPLE = 16
TEMP = 0.1
EPS = 1e-7
BQ = 128

_INF = 3.0e38


def _body(pb_ref, paT_ref, fb_ref, faT_ref, labc_ref, labr_ref,
          ls_ref, ms_ref):
    i = pl.program_id(0)

    pb = pb_ref[...]
    paT = paT_ref[...]

    mm = jnp.dot(pb.astype(jnp.bfloat16), paT.astype(jnp.bfloat16),
                 preferred_element_type=jnp.float32)
    d2 = (jnp.sum(pb * pb, axis=1, keepdims=True)
          + jnp.sum(paT * paT, axis=0, keepdims=True)
          - 2.0 * mm)

    nch = 16
    w = N // nch
    chunks = [d2[:, c * w:(c + 1) * w] for c in range(nch)]
    m1 = chunks[0]
    for c in chunks[1:]:
        m1 = jnp.minimum(m1, c)
    m2 = jnp.full((BQ, w), _INF, jnp.float32)
    for c in chunks:
        m2 = jnp.minimum(m2, jnp.where(c > m1, c, _INF))
    red = jnp.concatenate([m1, m2], axis=1)
    t0 = jnp.min(m1, axis=1, keepdims=True)
    t = t0
    for _ in range(NSAMPLE):
        t = jnp.min(jnp.where(red > t, red, _INF), axis=1, keepdims=True)
    nmask = jnp.logical_and(d2 <= t, d2 > t0)

    fb = fb_ref[...]
    faT = faT_ref[...]
    fb_hi = fb.astype(jnp.bfloat16)
    fb_lo = (fb - fb_hi.astype(jnp.float32)).astype(jnp.bfloat16)
    fa_hi = faT.astype(jnp.bfloat16)
    fa_lo = (faT - fa_hi.astype(jnp.float32)).astype(jnp.bfloat16)
    fmm = (jnp.dot(fb_hi, fa_hi, preferred_element_type=jnp.float32)
           + jnp.dot(fb_hi, fa_lo, preferred_element_type=jnp.float32)
           + jnp.dot(fb_lo, fa_hi, preferred_element_type=jnp.float32))
    fd2 = (jnp.sum(fb * fb, axis=1, keepdims=True)
           + jnp.sum(faT * faT, axis=0, keepdims=True)
           - 2.0 * fmm)
    row = i * BQ + jax.lax.broadcasted_iota(jnp.int32, (BQ, N), 0)
    col = jax.lax.broadcasted_iota(jnp.int32, (BQ, N), 1)
    fd2 = jnp.where(row == col, 0.0, fd2)
    x = jnp.where(nmask, jnp.maximum(fd2, 0.0) + EPS, 1.0e12)
    xmin = jnp.min(x, axis=1, keepdims=True)
    dist = jnp.sqrt(x)
    dmin = jnp.sqrt(xmin)
    e = jnp.exp((dmin - dist) * (1.0 / TEMP))
    eq = labc_ref[...] == labr_ref[...]

    pos = jnp.sum(jnp.where(eq, e, 0.0), axis=1, keepdims=True)
    neg = jnp.sum(e, axis=1, keepdims=True)
    pcnt = jnp.sum(jnp.where(jnp.logical_and(eq, nmask), 1.0, 0.0),
                   axis=1, keepdims=True)
    pm = jnp.logical_and(pcnt > 0.5, pcnt < NSAMPLE - 0.5).astype(jnp.float32)
    lpp = -jnp.log(pos / neg + EPS)

    pls = jnp.sum(lpp * pm, axis=0, keepdims=True)
    pms = jnp.sum(pm, axis=0, keepdims=True)

    @pl.when(i == 0)
    def _():
        ls_ref[...] = pls
        ms_ref[...] = pms

    @pl.when(i > 0)
    def _():
        ls_ref[...] += pls
        ms_ref[...] += pms


def kernel(p, features, labels):
    p = p.astype(jnp.float32)
    features = features.astype(jnp.float32)
    pp = jnp.pad(p, ((0, 0), (0, 5)))
    paT = pp.T
    faT = features.T
    lab = labels.astype(jnp.float32)
    labc = lab.reshape(N, 1)
    labr = lab.reshape(1, N)

    ls, ms = pl.pallas_call(
        _body,
        grid=(N // BQ,),
        in_specs=[
            pl.BlockSpec((BQ, 8), lambda i: (i, 0)),
            pl.BlockSpec((8, N), lambda i: (0, 0)),
            pl.BlockSpec((BQ, D), lambda i: (i, 0)),
            pl.BlockSpec((D, N), lambda i: (0, 0)),
            pl.BlockSpec((BQ, 1), lambda i: (i, 0)),
            pl.BlockSpec((1, N), lambda i: (0, 0)),
        ],
        out_specs=[
            pl.BlockSpec((1, 1), lambda i: (0, 0)),
            pl.BlockSpec((1, 1), lambda i: (0, 0)),
        ],
        out_shape=[
            jax.ShapeDtypeStruct((1, 1), jnp.float32),
            jax.ShapeDtypeStruct((1, 1), jnp.float32),
        ],
        interpret=_INTERPRET,
    )(pp, paT, features, faT, labc, labr)

    return (ls[0, 0] / jnp.maximum(ms[0, 0], 1.0)).astype(jnp.float32)


_INTERPRET = False


# speedup vs baseline: 43.9960x; 1.1712x over previous
import jax
import jax.numpy as jnp
from jax.experimental import pallas as pl

N = 8192
D = 32
NSAMPLE = 16
TEMP = 0.1
EPS = 1e-7
BQ = 256

_INF = 3.0e38


def _body(pb_ref, paT_ref, fb_ref, faT_ref, labc_ref, labr_ref,
          ls_ref, ms_ref):
    i = pl.program_id(0)

    pb = pb_ref[...]
    paT = paT_ref[...]

    mm = jnp.dot(pb.astype(jnp.bfloat16), paT.astype(jnp.bfloat16),
                 preferred_element_type=jnp.float32)
    d2 = (jnp.sum(pb * pb, axis=1, keepdims=True)
          + jnp.sum(paT * paT, axis=0, keepdims=True)
          - 2.0 * mm)

    nch = 32
    w = N // nch
    chunks = [d2[:, c * w:(c + 1) * w] for c in range(nch)]
    m1 = chunks[0]
    for c in chunks[1:]:
        m1 = jnp.minimum(m1, c)
    m2 = jnp.full((BQ, w), _INF, jnp.float32)
    for c in chunks:
        m2 = jnp.minimum(m2, jnp.where(c > m1, c, _INF))
    red = jnp.concatenate([m1, m2], axis=1)
    t0 = jnp.min(m1, axis=1, keepdims=True)
    t = t0
    for _ in range(NSAMPLE):
        t = jnp.min(jnp.where(red > t, red, _INF), axis=1, keepdims=True)
    nmask = jnp.logical_and(d2 <= t, d2 > t0)

    fb = fb_ref[...]
    faT = faT_ref[...]
    fb_hi = fb.astype(jnp.bfloat16)
    fb_lo = (fb - fb_hi.astype(jnp.float32)).astype(jnp.bfloat16)
    fa_hi = faT.astype(jnp.bfloat16)
    fa_lo = (faT - fa_hi.astype(jnp.float32)).astype(jnp.bfloat16)
    fmm = (jnp.dot(fb_hi, fa_hi, preferred_element_type=jnp.float32)
           + jnp.dot(fb_hi, fa_lo, preferred_element_type=jnp.float32)
           + jnp.dot(fb_lo, fa_hi, preferred_element_type=jnp.float32))
    s = 1.0 / (TEMP * TEMP)
    fd2 = (s * jnp.sum(fb * fb, axis=1, keepdims=True)
           + s * jnp.sum(faT * faT, axis=0, keepdims=True)
           - (2.0 * s) * fmm)
    x = jnp.where(nmask, jnp.maximum(fd2, 0.0) + EPS * s, 1.0e12)
    xmin = jnp.min(x, axis=1, keepdims=True)
    dist = jnp.sqrt(x)
    dmin = jnp.sqrt(xmin)
    e = jnp.exp(dmin - dist)
    eq = labc_ref[...] == labr_ref[...]

    pos = jnp.sum(jnp.where(eq, e, 0.0), axis=1, keepdims=True)
    neg = jnp.sum(e, axis=1, keepdims=True)
    pcnt = jnp.sum(jnp.where(jnp.logical_and(eq, nmask), 1.0, 0.0),
                   axis=1, keepdims=True)
    pm = jnp.logical_and(pcnt > 0.5, pcnt < NSAMPLE - 0.5).astype(jnp.float32)
    lpp = -jnp.log(pos / neg + EPS)

    pls = jnp.sum(lpp * pm, axis=0, keepdims=True)
    pms = jnp.sum(pm, axis=0, keepdims=True)

    @pl.when(i == 0)
    def _():
        ls_ref[...] = pls
        ms_ref[...] = pms

    @pl.when(i > 0)
    def _():
        ls_ref[...] += pls
        ms_ref[...] += pms


def kernel(p, features, labels):
    p = p.astype(jnp.float32)
    features = features.astype(jnp.float32)
    pp = jnp.pad(p, ((0, 0), (0, 5)))
    paT = pp.T
    faT = features.T
    lab = labels.astype(jnp.float32)
    labc = lab.reshape(N, 1)
    labr = lab.reshape(1, N)

    ls, ms = pl.pallas_call(
        _body,
        grid=(N // BQ,),
        in_specs=[
            pl.BlockSpec((BQ, 8), lambda i: (i, 0)),
            pl.BlockSpec((8, N), lambda i: (0, 0)),
            pl.BlockSpec((BQ, D), lambda i: (i, 0)),
            pl.BlockSpec((D, N), lambda i: (0, 0)),
            pl.BlockSpec((BQ, 1), lambda i: (i, 0)),
            pl.BlockSpec((1, N), lambda i: (0, 0)),
        ],
        out_specs=[
            pl.BlockSpec((1, 1), lambda i: (0, 0)),
            pl.BlockSpec((1, 1), lambda i: (0, 0)),
        ],
        out_shape=[
            jax.ShapeDtypeStruct((1, 1), jnp.float32),
            jax.ShapeDtypeStruct((1, 1), jnp.float32),
        ],
        interpret=_INTERPRET,
    )(pp, paT, features, faT, labc, labr)

    return (ls[0, 0] / jnp.maximum(ms[0, 0], 1.0)).astype(jnp.float32)


_INTERPRET = False


# speedup vs baseline: 48.7371x; 1.1078x over previous
import jax
import jax.numpy as jnp
from jax.experimental import pallas as pl

N = 8192
D = 32
NSAMPLE = 16
TEMP = 0.1
EPS = 1e-7
BQ = 256

_INF = 3.0e38


def _body(pb_ref, paT_ref, fb_ref, faT_ref, labc_ref, labr_ref,
          ls_ref, ms_ref):
    i = pl.program_id(0)

    pb = pb_ref[...]
    paT = paT_ref[...]

    mm = jnp.dot(pb.astype(jnp.bfloat16), paT.astype(jnp.bfloat16),
                 preferred_element_type=jnp.float32)
    d2 = (jnp.sum(pb * pb, axis=1, keepdims=True)
          + jnp.sum(paT * paT, axis=0, keepdims=True)
          - 2.0 * mm)

    nch = 32
    w = N // nch
    chunks = [d2[:, c * w:(c + 1) * w] for c in range(nch)]
    m1 = chunks[0]
    for c in chunks[1:]:
        m1 = jnp.minimum(m1, c)
    m2 = jnp.full((BQ, w), _INF, jnp.float32)
    for c in chunks:
        m2 = jnp.minimum(m2, jnp.where(c > m1, c, _INF))
    red = jnp.concatenate([m1, m2], axis=1)
    t0 = jnp.min(m1, axis=1, keepdims=True)
    t = t0
    for _ in range(NSAMPLE):
        t = jnp.min(jnp.where(red > t, red, _INF), axis=1, keepdims=True)
    nmask = jnp.logical_and(d2 <= t, d2 > t0)

    fb = fb_ref[...]
    faT = faT_ref[...]
    fb_hi = fb.astype(jnp.bfloat16)
    fb_lo = (fb - fb_hi.astype(jnp.float32)).astype(jnp.bfloat16)
    fa_hi = faT.astype(jnp.bfloat16)
    fa_lo = (faT - fa_hi.astype(jnp.float32)).astype(jnp.bfloat16)
    fmm = (jnp.dot(fb_hi, fa_hi, preferred_element_type=jnp.float32)
           + jnp.dot(fb_hi, fa_lo, preferred_element_type=jnp.float32)
           + jnp.dot(fb_lo, fa_hi, preferred_element_type=jnp.float32))
    log2e = 1.4426950408889634
    s = (log2e / TEMP) ** 2
    fd2 = (s * jnp.sum(fb * fb, axis=1, keepdims=True)
           + s * jnp.sum(faT * faT, axis=0, keepdims=True)
           - (2.0 * s) * fmm)
    x = jnp.where(nmask, jnp.maximum(fd2, 0.0) + EPS * s, 1.0e12)
    xmin = jnp.min(x, axis=1, keepdims=True)
    dist = x * jax.lax.rsqrt(x)
    dmin = jnp.sqrt(xmin)
    e = jnp.exp2(dmin - dist)
    eq = labc_ref[...] == labr_ref[...]

    pos = jnp.sum(jnp.where(eq, e, 0.0), axis=1, keepdims=True)
    neg = jnp.sum(e, axis=1, keepdims=True)
    pcnt = jnp.sum(jnp.where(jnp.logical_and(eq, nmask), 1.0, 0.0),
                   axis=1, keepdims=True)
    pm = jnp.logical_and(pcnt > 0.5, pcnt < NSAMPLE - 0.5).astype(jnp.float32)
    lpp = -jnp.log(pos / neg + EPS)

    pls = jnp.sum(lpp * pm, axis=0, keepdims=True)
    pms = jnp.sum(pm, axis=0, keepdims=True)

    @pl.when(i == 0)
    def _():
        ls_ref[...] = pls
        ms_ref[...] = pms

    @pl.when(i > 0)
    def _():
        ls_ref[...] += pls
        ms_ref[...] += pms


def kernel(p, features, labels):
    p = p.astype(jnp.float32)
    features = features.astype(jnp.float32)
    pp = jnp.pad(p, ((0, 0), (0, 5)))
    paT = pp.T
    faT = features.T
    lab = labels.astype(jnp.float32)
    labc = lab.reshape(N, 1)
    labr = lab.reshape(1, N)

    ls, ms = pl.pallas_call(
        _body,
        grid=(N // BQ,),
        in_specs=[
            pl.BlockSpec((BQ, 8), lambda i: (i, 0)),
            pl.BlockSpec((8, N), lambda i: (0, 0)),
            pl.BlockSpec((BQ, D), lambda i: (i, 0)),
            pl.BlockSpec((D, N), lambda i: (0, 0)),
            pl.BlockSpec((BQ, 1), lambda i: (i, 0)),
            pl.BlockSpec((1, N), lambda i: (0, 0)),
        ],
        out_specs=[
            pl.BlockSpec((1, 1), lambda i: (0, 0)),
            pl.BlockSpec((1, 1), lambda i: (0, 0)),
        ],
        out_shape=[
            jax.ShapeDtypeStruct((1, 1), jnp.float32),
            jax.ShapeDtypeStruct((1, 1), jnp.float32),
        ],
        interpret=_INTERPRET,
    )(pp, paT, features, faT, labc, labr)

    return (ls[0, 0] / jnp.maximum(ms[0, 0], 1.0)).astype(jnp.float32)


_INTERPRET = False
